# baseline (speedup 1.0000x reference)
import functools

import jax
import jax.numpy as jnp
from jax import lax
from jax.experimental import pallas as pl
from jax.experimental.pallas import tpu as pltpu
from jax.experimental.pallas import tpu_sc as plsc

_EPS = 1e-5
_NC = 2
_NS = 16
_NW = _NC * _NS
_K = 64
_CPW = 160
_NBUF = 2


def _deg_sc(dst2d, n, npad):
    mesh = plsc.VectorSubcoreMesh(core_axis_name="c", subcore_axis_name="s")

    @functools.partial(
        pl.kernel,
        out_type=jax.ShapeDtypeStruct((_NC * npad,), jnp.float32),
        mesh=mesh,
        scratch_types=[
            pltpu.VMEM((_CPW, _K), jnp.int32),
            pltpu.VMEM((_K,), jnp.float32),
            pltpu.VMEM((npad,), jnp.float32),
            pltpu.VMEM_SHARED((npad,), jnp.float32),
            pltpu.SemaphoreType.DMA,
        ],
    )
    def body(dst_hbm, out_hbm, didx, ones, zbuf, acc, sem):
        cid = lax.axis_index("c")
        sid = lax.axis_index("s")
        wid = sid * _NC + cid
        pltpu.sync_copy(dst_hbm.at[pl.ds(wid * _CPW, _CPW)], didx)
        for j in range(_K // 16):
            ones[pl.ds(j * 16, 16)] = jnp.full((16,), 1.0, jnp.float32)

        @pl.when(sid == 0)
        def _zero():
            def zstep(i, c):
                zbuf[pl.ds(i * 16, 16)] = jnp.zeros((16,), jnp.float32)
                return c
            lax.fori_loop(0, npad // 16, zstep, 0)
            pltpu.sync_copy(zbuf, acc)

        plsc.subcore_barrier()

        def step(i, c):
            pltpu.async_copy(ones, acc.at[didx.at[i]], sem, add=True)
            return c

        lax.fori_loop(0, _CPW, step, 0)

        def drain(i, c):
            pltpu.make_async_copy(ones, acc.at[didx.at[0]], sem).wait()
            return c

        lax.fori_loop(0, _CPW, drain, 0)
        plsc.subcore_barrier()

        @pl.when(sid == 0)
        def _out():
            pltpu.sync_copy(acc, out_hbm.at[pl.ds(cid * npad, npad)])

    return body(dst2d)


def _seg_sum_sc(g, src2d, dst2d, zeros):
    n, d = g.shape
    nacc = zeros.shape[0]
    rpt = (n // _NS) // 8 * 8
    ztail = nacc - _NS * rpt
    otail = n - _NS * rpt
    epw = _CPW * _K
    mesh = plsc.VectorSubcoreMesh(core_axis_name="c", subcore_axis_name="s")

    @functools.partial(
        pl.kernel,
        out_type=jax.ShapeDtypeStruct((_NC, n, d), jnp.float32),
        mesh=mesh,
        scratch_types=(
            [pltpu.VMEM_SHARED((nacc, d), jnp.float32)]
            + [pltpu.VMEM((_K,), jnp.int32) for _ in range(2 * _NBUF)]
            + [pltpu.VMEM((_K, d), jnp.float32) for _ in range(_NBUF)]
            + [pltpu.SemaphoreType.DMA for _ in range(_NBUF)]
        ),
    )
    def body(g_hbm, src_hbm, dst_hbm, z_hbm, out_hbm, acc, *bufs):
        sidx = bufs[:_NBUF]
        didx = bufs[_NBUF:2 * _NBUF]
        rows = bufs[2 * _NBUF:3 * _NBUF]
        gsem = bufs[3 * _NBUF:]
        cid = lax.axis_index("c")
        sid = lax.axis_index("s")
        wid = sid * _NC + cid
        base = wid * epw

        for b in range(_NBUF):
            pltpu.sync_copy(src_hbm.at[pl.ds(base + b * _K, _K)], sidx[b])
            pltpu.sync_copy(dst_hbm.at[pl.ds(base + b * _K, _K)], didx[b])
            pltpu.async_copy(g_hbm.at[sidx[b]], rows[b], gsem[b])
        pltpu.sync_copy(z_hbm.at[pl.ds(sid * rpt, rpt)],
                        acc.at[pl.ds(sid * rpt, rpt)])

        @pl.when(sid == 0)
        def _ztail():
            pltpu.sync_copy(z_hbm.at[pl.ds(_NS * rpt, ztail)],
                            acc.at[pl.ds(_NS * rpt, ztail)])

        plsc.subcore_barrier()

        nblk = _CPW // _NBUF

        def blk(k, c):
            j0 = k * _NBUF
            for b in range(_NBUF):
                j = j0 + b
                pltpu.make_async_copy(g_hbm.at[sidx[b]], rows[b],
                                      gsem[b]).wait()
                pltpu.sync_copy(rows[b], acc.at[didx[b]], add=True)

                @pl.when(j + _NBUF < _CPW)
                def _next():
                    nxt = base + (j + _NBUF) * _K
                    pltpu.sync_copy(src_hbm.at[pl.ds(nxt, _K)], sidx[b])
                    pltpu.sync_copy(dst_hbm.at[pl.ds(nxt, _K)], didx[b])
                    pltpu.async_copy(g_hbm.at[sidx[b]], rows[b], gsem[b])
            return c

        lax.fori_loop(0, nblk, blk, 0)
        plsc.subcore_barrier()

        pltpu.sync_copy(acc.at[pl.ds(sid * rpt, rpt)],
                        out_hbm.at[cid, pl.ds(sid * rpt, rpt)])

        @pl.when(sid == 0)
        def _otail():
            pltpu.sync_copy(acc.at[pl.ds(_NS * rpt, otail)],
                            out_hbm.at[cid, pl.ds(_NS * rpt, otail)])

    return body(g, src2d, dst2d, zeros)


def _tc1(x, w1, deg_t):
    n, d = x.shape

    def body(x_ref, w_ref, deg_ref, g1_ref, dinv_ref):
        deg = deg_ref[:, 0:1] + deg_ref[:, 1:2] + 1.0
        dinv = lax.rsqrt(deg)
        h = jnp.dot(x_ref[...], w_ref[...], preferred_element_type=jnp.float32)
        g1_ref[...] = h * dinv
        dinv_ref[...] = dinv

    return pl.pallas_call(
        body,
        out_shape=(jax.ShapeDtypeStruct((n, d), jnp.float32),
                   jax.ShapeDtypeStruct((n, 1), jnp.float32)),
    )(x, w1, deg_t)


def _tc2(s1p, g1, dinv, b1, gnw, gnb, gna, w2):
    n, d = g1.shape

    def body(sp_ref, g_ref, di_ref, b_ref, w_ref, bt_ref, a_ref, w2_ref,
             x1_ref, g2_ref):
        s = sp_ref[0] + sp_ref[1] + g_ref[...]
        y = di_ref[...] * s + b_ref[...]
        mean = jnp.mean(y, axis=0, keepdims=True)
        o = y - a_ref[...] * mean
        var = jnp.mean(o * o, axis=0, keepdims=True)
        x1 = jnp.maximum(w_ref[...] * o * lax.rsqrt(var + _EPS) + bt_ref[...],
                         0.0)
        x1_ref[...] = x1
        g2_ref[...] = jnp.dot(x1, w2_ref[...],
                              preferred_element_type=jnp.float32) * di_ref[...]

    return pl.pallas_call(
        body,
        out_shape=(jax.ShapeDtypeStruct((n, d), jnp.float32),
                   jax.ShapeDtypeStruct((n, d), jnp.float32)),
    )(s1p, g1, dinv, b1, gnw, gnb, gna, w2)


def _tc3(s2p, g2, dinv, b2, gnw, gnb, gna, x1, wr, br):
    n, d = g2.shape

    def body(sp_ref, g_ref, di_ref, b_ref, w_ref, bt_ref, a_ref, x1_ref,
             wr_ref, br_ref, out_ref):
        s = sp_ref[0] + sp_ref[1] + g_ref[...]
        y = di_ref[...] * s + b_ref[...]
        mean = jnp.mean(y, axis=0, keepdims=True)
        o = y - a_ref[...] * mean
        var = jnp.mean(o * o, axis=0, keepdims=True)
        x2 = jnp.maximum(w_ref[...] * o * lax.rsqrt(var + _EPS) + bt_ref[...],
                         0.0)
        out_ref[...] = jnp.dot(x1_ref[...] + x2, wr_ref[...],
                               preferred_element_type=jnp.float32) + br_ref[...]

    return pl.pallas_call(
        body,
        out_shape=jax.ShapeDtypeStruct((n, d), jnp.float32),
    )(s2p, g2, dinv, b2, gnw, gnb, gna, x1, wr, br)


def kernel(x, edge_index, W1, b1, W2, b2, gn1_w, gn1_b, gn1_a, gn2_w, gn2_b,
           gn2_a, Wr, br):
    n, d = x.shape
    e = edge_index.shape[1]
    epad = _NW * _CPW * _K
    assert e <= epad and n % 16 == 0

    src = edge_index[0]
    dst = edge_index[1]
    pad = epad - e
    dummy = n + (jnp.arange(pad, dtype=jnp.int32) % 128)
    srcp = jnp.concatenate([src, jnp.zeros((pad,), jnp.int32)])
    dstp = jnp.concatenate([dst, dummy])
    dst2d = dstp.reshape(epad // _K, _K)
    zeros = jnp.zeros((n + 128, d), jnp.float32)

    npad = -(-(n + 128) // 128) * 128
    degp = _deg_sc(dst2d, n, npad).reshape(_NC, npad)[:, :n]
    deg_t = degp.T

    g1, dinv = _tc1(x, W1, deg_t)
    s1p = _seg_sum_sc(g1, srcp, dstp, zeros)
    x1, g2 = _tc2(s1p, g1, dinv, b1.reshape(1, d), gn1_w.reshape(1, d),
                  gn1_b.reshape(1, d), gn1_a.reshape(1, d), W2)
    s2p = _seg_sum_sc(g2, srcp, dstp, zeros)
    return _tc3(s2p, g2, dinv, b2.reshape(1, d), gn2_w.reshape(1, d),
                gn2_b.reshape(1, d), gn2_a.reshape(1, d), x1, Wr, br.reshape(1, d))


# speedup vs baseline: 2.6472x; 2.6472x over previous
import functools

import jax
import jax.numpy as jnp
from jax import lax
from jax.experimental import pallas as pl
from jax.experimental.pallas import tpu as pltpu
from jax.experimental.pallas import tpu_sc as plsc

_EPS = 1e-5
_NC = 2
_NS = 16
_NW = _NC * _NS
_K = 128
_NBUF = 2


def _deg_sc(dst, n, npad):
    e = dst.shape[0]
    epw = e // _NW
    nch = epw // _K
    tail = epw - nch * _K
    mesh = plsc.VectorSubcoreMesh(core_axis_name="c", subcore_axis_name="s")

    @functools.partial(
        pl.kernel,
        out_type=jax.ShapeDtypeStruct((_NC * npad,), jnp.float32),
        mesh=mesh,
        scratch_types=[
            pltpu.VMEM((_K,), jnp.int32),
            pltpu.VMEM((16,), jnp.int32),
            pltpu.VMEM((_K,), jnp.float32),
            pltpu.VMEM((npad,), jnp.float32),
            pltpu.VMEM_SHARED((npad,), jnp.float32),
            pltpu.SemaphoreType.DMA,
        ],
    )
    def body(dst_hbm, out_hbm, didx, didx_t, ones, zbuf, acc, sem):
        cid = lax.axis_index("c")
        sid = lax.axis_index("s")
        wid = sid * _NC + cid
        base = wid * epw
        for j in range(_K // 16):
            ones[pl.ds(j * 16, 16)] = jnp.full((16,), 1.0, jnp.float32)

        @pl.when(sid == 0)
        def _zero():
            def zstep(i, c):
                zbuf[pl.ds(i * 16, 16)] = jnp.zeros((16,), jnp.float32)
                return c
            lax.fori_loop(0, npad // 16, zstep, 0)
            pltpu.sync_copy(zbuf, acc)

        plsc.subcore_barrier()

        def step(i, c):
            pltpu.sync_copy(dst_hbm.at[pl.ds(base + i * _K, _K)], didx)
            pltpu.sync_copy(ones, acc.at[didx], add=True)
            return c

        lax.fori_loop(0, nch, step, 0)
        if tail:
            pltpu.sync_copy(dst_hbm.at[pl.ds(base + nch * _K, tail)], didx_t)
            pltpu.sync_copy(ones.at[pl.ds(0, tail)], acc.at[didx_t], add=True)
        plsc.subcore_barrier()

        @pl.when(sid == 0)
        def _out():
            pltpu.sync_copy(acc, out_hbm.at[pl.ds(cid * npad, npad)])

    return body(dst)


def _seg_sum_sc(g, src, dst, zeros):
    n, d = g.shape
    e = src.shape[0]
    epw = e // _NW
    nch = epw // _K
    tail = epw - nch * _K
    assert tail % 8 == 0 and tail <= 16
    rpt = (n // _NS) // 8 * 8
    otail = n - _NS * rpt
    mesh = plsc.VectorSubcoreMesh(core_axis_name="c", subcore_axis_name="s")

    @functools.partial(
        pl.kernel,
        out_type=jax.ShapeDtypeStruct((_NC, n, d), jnp.float32),
        mesh=mesh,
        scratch_types=(
            [pltpu.VMEM_SHARED((n, d), jnp.float32),
             pltpu.VMEM((16,), jnp.int32),
             pltpu.VMEM((16,), jnp.int32),
             pltpu.VMEM((16, d), jnp.float32)]
            + [pltpu.VMEM((_K,), jnp.int32) for _ in range(2 * _NBUF)]
            + [pltpu.VMEM((_K, d), jnp.float32) for _ in range(_NBUF)]
            + [pltpu.SemaphoreType.DMA for _ in range(_NBUF)]
        ),
    )
    def body(g_hbm, src_hbm, dst_hbm, z_hbm, out_hbm, acc, sidx_t, didx_t,
             rows_t, *bufs):
        sidx = bufs[:_NBUF]
        didx = bufs[_NBUF:2 * _NBUF]
        rows = bufs[2 * _NBUF:3 * _NBUF]
        gsem = bufs[3 * _NBUF:]
        cid = lax.axis_index("c")
        sid = lax.axis_index("s")
        wid = sid * _NC + cid
        base = wid * epw

        for b in range(_NBUF):
            pltpu.sync_copy(src_hbm.at[pl.ds(base + b * _K, _K)], sidx[b])
            pltpu.sync_copy(dst_hbm.at[pl.ds(base + b * _K, _K)], didx[b])
            pltpu.async_copy(g_hbm.at[sidx[b]], rows[b], gsem[b])
        pltpu.sync_copy(z_hbm.at[pl.ds(sid * rpt, rpt)],
                        acc.at[pl.ds(sid * rpt, rpt)])

        @pl.when(sid == 0)
        def _ztail():
            pltpu.sync_copy(z_hbm.at[pl.ds(_NS * rpt, otail)],
                            acc.at[pl.ds(_NS * rpt, otail)])

        plsc.subcore_barrier()

        nblk = nch // _NBUF

        def blk(k, c):
            j0 = k * _NBUF
            for b in range(_NBUF):
                j = j0 + b
                pltpu.make_async_copy(g_hbm.at[sidx[b]], rows[b],
                                      gsem[b]).wait()
                pltpu.sync_copy(rows[b], acc.at[didx[b]], add=True)

                @pl.when(j + _NBUF < nch)
                def _next():
                    nxt = base + (j + _NBUF) * _K
                    pltpu.sync_copy(src_hbm.at[pl.ds(nxt, _K)], sidx[b])
                    pltpu.sync_copy(dst_hbm.at[pl.ds(nxt, _K)], didx[b])
                    pltpu.async_copy(g_hbm.at[sidx[b]], rows[b], gsem[b])
            return c

        lax.fori_loop(0, nblk, blk, 0)
        if tail:
            tbase = base + nch * _K
            pltpu.sync_copy(src_hbm.at[pl.ds(tbase, tail)], sidx_t)
            pltpu.sync_copy(dst_hbm.at[pl.ds(tbase, tail)], didx_t)
            pltpu.sync_copy(g_hbm.at[sidx_t], rows_t)
            pltpu.sync_copy(rows_t, acc.at[didx_t], add=True)
        plsc.subcore_barrier()

        pltpu.sync_copy(acc.at[pl.ds(sid * rpt, rpt)],
                        out_hbm.at[cid, pl.ds(sid * rpt, rpt)])

        @pl.when(sid == 0)
        def _otail():
            pltpu.sync_copy(acc.at[pl.ds(_NS * rpt, otail)],
                            out_hbm.at[cid, pl.ds(_NS * rpt, otail)])

    return body(g, src, dst, zeros)


def _tc1(x, w1, deg_t):
    n, d = x.shape

    def body(x_ref, w_ref, deg_ref, g1_ref, dinv_ref):
        deg = deg_ref[:, 0:1] + deg_ref[:, 1:2] + 1.0
        dinv = lax.rsqrt(deg)
        h = jnp.dot(x_ref[...], w_ref[...], preferred_element_type=jnp.float32)
        g1_ref[...] = h * dinv
        dinv_ref[...] = dinv

    return pl.pallas_call(
        body,
        out_shape=(jax.ShapeDtypeStruct((n, d), jnp.float32),
                   jax.ShapeDtypeStruct((n, 1), jnp.float32)),
    )(x, w1, deg_t)


def _tc2(s1p, g1, dinv, b1, gnw, gnb, gna, w2):
    n, d = g1.shape

    def body(sp_ref, g_ref, di_ref, b_ref, w_ref, bt_ref, a_ref, w2_ref,
             x1_ref, g2_ref):
        s = sp_ref[0] + sp_ref[1] + g_ref[...]
        y = di_ref[...] * s + b_ref[...]
        mean = jnp.mean(y, axis=0, keepdims=True)
        o = y - a_ref[...] * mean
        var = jnp.mean(o * o, axis=0, keepdims=True)
        x1 = jnp.maximum(w_ref[...] * o * lax.rsqrt(var + _EPS) + bt_ref[...],
                         0.0)
        x1_ref[...] = x1
        g2_ref[...] = jnp.dot(x1, w2_ref[...],
                              preferred_element_type=jnp.float32) * di_ref[...]

    return pl.pallas_call(
        body,
        out_shape=(jax.ShapeDtypeStruct((n, d), jnp.float32),
                   jax.ShapeDtypeStruct((n, d), jnp.float32)),
    )(s1p, g1, dinv, b1, gnw, gnb, gna, w2)


def _tc3(s2p, g2, dinv, b2, gnw, gnb, gna, x1, wr, br):
    n, d = g2.shape

    def body(sp_ref, g_ref, di_ref, b_ref, w_ref, bt_ref, a_ref, x1_ref,
             wr_ref, br_ref, out_ref):
        s = sp_ref[0] + sp_ref[1] + g_ref[...]
        y = di_ref[...] * s + b_ref[...]
        mean = jnp.mean(y, axis=0, keepdims=True)
        o = y - a_ref[...] * mean
        var = jnp.mean(o * o, axis=0, keepdims=True)
        x2 = jnp.maximum(w_ref[...] * o * lax.rsqrt(var + _EPS) + bt_ref[...],
                         0.0)
        out_ref[...] = jnp.dot(x1_ref[...] + x2, wr_ref[...],
                               preferred_element_type=jnp.float32) + br_ref[...]

    return pl.pallas_call(
        body,
        out_shape=jax.ShapeDtypeStruct((n, d), jnp.float32),
    )(s2p, g2, dinv, b2, gnw, gnb, gna, x1, wr, br)


def kernel(x, edge_index, W1, b1, W2, b2, gn1_w, gn1_b, gn1_a, gn2_w, gn2_b,
           gn2_a, Wr, br):
    n, d = x.shape
    e = edge_index.shape[1]
    assert e % _NW == 0 and n % 16 == 0

    src = edge_index[0]
    dst = edge_index[1]
    zeros = jnp.zeros((n, d), jnp.float32)

    npad = -(-n // 128) * 128
    degp = _deg_sc(dst, n, npad).reshape(_NC, npad)[:, :n]
    deg_t = degp.T

    g1, dinv = _tc1(x, W1, deg_t)
    s1p = _seg_sum_sc(g1, src, dst, zeros)
    x1, g2 = _tc2(s1p, g1, dinv, b1.reshape(1, d), gn1_w.reshape(1, d),
                  gn1_b.reshape(1, d), gn1_a.reshape(1, d), W2)
    s2p = _seg_sum_sc(g2, src, dst, zeros)
    return _tc3(s2p, g2, dinv, b2.reshape(1, d), gn2_w.reshape(1, d),
                gn2_b.reshape(1, d), gn2_a.reshape(1, d), x1, Wr, br.reshape(1, d))


# speedup vs baseline: 3.2491x; 1.2274x over previous
import functools

import jax
import jax.numpy as jnp
from jax import lax
from jax.experimental import pallas as pl
from jax.experimental.pallas import tpu as pltpu
from jax.experimental.pallas import tpu_sc as plsc

_EPS = 1e-5
_NC = 2
_NS = 16
_NW = _NC * _NS
_K = 128
_NBUF = 2
_NIDX = 4


def _deg_sc(dst, n, npad):
    e = dst.shape[0]
    epw = e // _NW
    nch = epw // _K
    tail = epw - nch * _K
    mesh = plsc.VectorSubcoreMesh(core_axis_name="c", subcore_axis_name="s")

    @functools.partial(
        pl.kernel,
        out_type=jax.ShapeDtypeStruct((_NC * npad,), jnp.float32),
        mesh=mesh,
        scratch_types=[
            pltpu.VMEM((_K,), jnp.int32),
            pltpu.VMEM((16,), jnp.int32),
            pltpu.VMEM((_K,), jnp.float32),
            pltpu.VMEM((npad,), jnp.float32),
            pltpu.VMEM_SHARED((npad,), jnp.float32),
            pltpu.SemaphoreType.DMA,
        ],
    )
    def body(dst_hbm, out_hbm, didx, didx_t, ones, zbuf, acc, sem):
        cid = lax.axis_index("c")
        sid = lax.axis_index("s")
        wid = sid * _NC + cid
        base = wid * epw
        for j in range(_K // 16):
            ones[pl.ds(j * 16, 16)] = jnp.full((16,), 1.0, jnp.float32)

        @pl.when(sid == 0)
        def _zero():
            def zstep(i, c):
                zbuf[pl.ds(i * 16, 16)] = jnp.zeros((16,), jnp.float32)
                return c
            lax.fori_loop(0, npad // 16, zstep, 0)
            pltpu.sync_copy(zbuf, acc)

        plsc.subcore_barrier()

        def step(i, c):
            pltpu.sync_copy(dst_hbm.at[pl.ds(base + i * _K, _K)], didx)
            pltpu.sync_copy(ones, acc.at[didx], add=True)
            return c

        lax.fori_loop(0, nch, step, 0)
        if tail:
            pltpu.sync_copy(dst_hbm.at[pl.ds(base + nch * _K, tail)], didx_t)
            pltpu.sync_copy(ones.at[pl.ds(0, tail)], acc.at[didx_t], add=True)
        plsc.subcore_barrier()

        @pl.when(sid == 0)
        def _out():
            pltpu.sync_copy(acc, out_hbm.at[pl.ds(cid * npad, npad)])

    return body(dst)


def _seg_sum_sc(g, src, dst, zeros):
    n, d = g.shape
    e = src.shape[0]
    epw = e // _NW
    nch = epw // _K
    tail = epw - nch * _K
    assert tail % 8 == 0 and tail <= 16
    rpt = (n // _NS) // 8 * 8
    otail = n - _NS * rpt
    mesh = plsc.VectorSubcoreMesh(core_axis_name="c", subcore_axis_name="s")

    @functools.partial(
        pl.kernel,
        out_type=jax.ShapeDtypeStruct((_NC, n, d), jnp.float32),
        mesh=mesh,
        scratch_types=(
            [pltpu.VMEM_SHARED((n, d), jnp.float32),
             pltpu.VMEM((16,), jnp.int32),
             pltpu.VMEM((16,), jnp.int32),
             pltpu.VMEM((16, d), jnp.float32)]
            + [pltpu.VMEM((_K,), jnp.int32) for _ in range(2 * _NIDX)]
            + [pltpu.VMEM((_K, d), jnp.float32) for _ in range(_NBUF)]
            + [pltpu.SemaphoreType.DMA for _ in range(_NBUF + _NIDX)]
        ),
    )
    def body(g_hbm, src_hbm, dst_hbm, z_hbm, out_hbm, acc, sidx_t, didx_t,
             rows_t, *bufs):
        sidx = bufs[:_NIDX]
        didx = bufs[_NIDX:2 * _NIDX]
        rows = bufs[2 * _NIDX:2 * _NIDX + _NBUF]
        gsem = bufs[2 * _NIDX + _NBUF:2 * _NIDX + 2 * _NBUF]
        isem = bufs[2 * _NIDX + 2 * _NBUF:]
        cid = lax.axis_index("c")
        sid = lax.axis_index("s")
        wid = sid * _NC + cid
        base = wid * epw

        for q in range(_NIDX):
            pltpu.sync_copy(src_hbm.at[pl.ds(base + q * _K, _K)], sidx[q])
            pltpu.sync_copy(dst_hbm.at[pl.ds(base + q * _K, _K)], didx[q])
        for b in range(_NBUF):
            pltpu.async_copy(g_hbm.at[sidx[b]], rows[b], gsem[b])
        pltpu.sync_copy(z_hbm.at[pl.ds(sid * rpt, rpt)],
                        acc.at[pl.ds(sid * rpt, rpt)])

        @pl.when(sid == 0)
        def _ztail():
            pltpu.sync_copy(z_hbm.at[pl.ds(_NS * rpt, otail)],
                            acc.at[pl.ds(_NS * rpt, otail)])

        plsc.subcore_barrier()

        nsb = nch // _NIDX
        rem = nch - nsb * _NIDX

        def chunk_step(j, q, b, prefetch_ok):
            pltpu.make_async_copy(g_hbm.at[sidx[q]], rows[b], gsem[b]).wait()
            pltpu.sync_copy(rows[b], acc.at[didx[q]], add=True)
            if prefetch_ok:
                @pl.when(j + _NIDX < nch)
                def _pfidx():
                    nxt = base + (j + _NIDX) * _K
                    pltpu.async_copy(src_hbm.at[pl.ds(nxt, _K)], sidx[q],
                                     isem[q])
                    pltpu.async_copy(dst_hbm.at[pl.ds(nxt, _K)], didx[q],
                                     isem[q])

                qn = (q + _NBUF) % _NIDX

                @pl.when(j + _NBUF < nch)
                def _next():
                    @pl.when(j + _NBUF >= _NIDX)
                    def _wait_idx():
                        nxt = base + (j + _NBUF) * _K
                        pltpu.make_async_copy(
                            src_hbm.at[pl.ds(nxt, _K)], sidx[qn],
                            isem[qn]).wait()
                        pltpu.make_async_copy(
                            dst_hbm.at[pl.ds(nxt, _K)], didx[qn],
                            isem[qn]).wait()

                    pltpu.async_copy(g_hbm.at[sidx[qn]], rows[b], gsem[b])

        def blk(k, c):
            j0 = k * _NIDX
            for q in range(_NIDX):
                chunk_step(j0 + q, q, q % _NBUF, True)
            return c

        lax.fori_loop(0, nsb, blk, 0)
        for r in range(rem):
            j = nsb * _NIDX + r
            chunk_step(j, j % _NIDX, j % _NBUF, False)
        if tail:
            tbase = base + nch * _K
            pltpu.sync_copy(src_hbm.at[pl.ds(tbase, tail)], sidx_t)
            pltpu.sync_copy(dst_hbm.at[pl.ds(tbase, tail)], didx_t)
            pltpu.sync_copy(g_hbm.at[sidx_t], rows_t)
            pltpu.sync_copy(rows_t, acc.at[didx_t], add=True)
        plsc.subcore_barrier()

        pltpu.sync_copy(acc.at[pl.ds(sid * rpt, rpt)],
                        out_hbm.at[cid, pl.ds(sid * rpt, rpt)])

        @pl.when(sid == 0)
        def _otail():
            pltpu.sync_copy(acc.at[pl.ds(_NS * rpt, otail)],
                            out_hbm.at[cid, pl.ds(_NS * rpt, otail)])

    return body(g, src, dst, zeros)


def _tc1(x, w1, deg_t):
    n, d = x.shape

    def body(x_ref, w_ref, deg_ref, g1_ref, dinv_ref):
        deg = deg_ref[:, 0:1] + deg_ref[:, 1:2] + 1.0
        dinv = lax.rsqrt(deg)
        h = jnp.dot(x_ref[...], w_ref[...], preferred_element_type=jnp.float32)
        g1_ref[...] = h * dinv
        dinv_ref[...] = dinv

    return pl.pallas_call(
        body,
        out_shape=(jax.ShapeDtypeStruct((n, d), jnp.float32),
                   jax.ShapeDtypeStruct((n, 1), jnp.float32)),
    )(x, w1, deg_t)


def _tc2(s1p, g1, dinv, b1, gnw, gnb, gna, w2):
    n, d = g1.shape

    def body(sp_ref, g_ref, di_ref, b_ref, w_ref, bt_ref, a_ref, w2_ref,
             x1_ref, g2_ref):
        s = sp_ref[0] + sp_ref[1] + g_ref[...]
        y = di_ref[...] * s + b_ref[...]
        mean = jnp.mean(y, axis=0, keepdims=True)
        o = y - a_ref[...] * mean
        var = jnp.mean(o * o, axis=0, keepdims=True)
        x1 = jnp.maximum(w_ref[...] * o * lax.rsqrt(var + _EPS) + bt_ref[...],
                         0.0)
        x1_ref[...] = x1
        g2_ref[...] = jnp.dot(x1, w2_ref[...],
                              preferred_element_type=jnp.float32) * di_ref[...]

    return pl.pallas_call(
        body,
        out_shape=(jax.ShapeDtypeStruct((n, d), jnp.float32),
                   jax.ShapeDtypeStruct((n, d), jnp.float32)),
    )(s1p, g1, dinv, b1, gnw, gnb, gna, w2)


def _tc3(s2p, g2, dinv, b2, gnw, gnb, gna, x1, wr, br):
    n, d = g2.shape

    def body(sp_ref, g_ref, di_ref, b_ref, w_ref, bt_ref, a_ref, x1_ref,
             wr_ref, br_ref, out_ref):
        s = sp_ref[0] + sp_ref[1] + g_ref[...]
        y = di_ref[...] * s + b_ref[...]
        mean = jnp.mean(y, axis=0, keepdims=True)
        o = y - a_ref[...] * mean
        var = jnp.mean(o * o, axis=0, keepdims=True)
        x2 = jnp.maximum(w_ref[...] * o * lax.rsqrt(var + _EPS) + bt_ref[...],
                         0.0)
        out_ref[...] = jnp.dot(x1_ref[...] + x2, wr_ref[...],
                               preferred_element_type=jnp.float32) + br_ref[...]

    return pl.pallas_call(
        body,
        out_shape=jax.ShapeDtypeStruct((n, d), jnp.float32),
    )(s2p, g2, dinv, b2, gnw, gnb, gna, x1, wr, br)


def kernel(x, edge_index, W1, b1, W2, b2, gn1_w, gn1_b, gn1_a, gn2_w, gn2_b,
           gn2_a, Wr, br):
    n, d = x.shape
    e = edge_index.shape[1]
    assert e % _NW == 0 and n % 16 == 0

    src = edge_index[0]
    dst = edge_index[1]
    zeros = jnp.zeros((n, d), jnp.float32)

    npad = -(-n // 128) * 128
    degp = _deg_sc(dst, n, npad).reshape(_NC, npad)[:, :n]
    deg_t = degp.T

    g1, dinv = _tc1(x, W1, deg_t)
    s1p = _seg_sum_sc(g1, src, dst, zeros)
    x1, g2 = _tc2(s1p, g1, dinv, b1.reshape(1, d), gn1_w.reshape(1, d),
                  gn1_b.reshape(1, d), gn1_a.reshape(1, d), W2)
    s2p = _seg_sum_sc(g2, src, dst, zeros)
    return _tc3(s2p, g2, dinv, b2.reshape(1, d), gn2_w.reshape(1, d),
                gn2_b.reshape(1, d), gn2_a.reshape(1, d), x1, Wr, br.reshape(1, d))


# speedup vs baseline: 3.6704x; 1.1297x over previous
import functools

import jax
import jax.numpy as jnp
from jax import lax
from jax.experimental import pallas as pl
from jax.experimental.pallas import tpu as pltpu
from jax.experimental.pallas import tpu_sc as plsc

_EPS = 1e-5
_NC = 2
_NS = 16
_NW = _NC * _NS
_K = 128
_NBUF = 2
_NIDX = 4


def _deg_sc(dst, n, npad):
    e = dst.shape[0]
    epw = e // _NW
    nch = epw // _K
    tail = epw - nch * _K
    mesh = plsc.VectorSubcoreMesh(core_axis_name="c", subcore_axis_name="s")

    @functools.partial(
        pl.kernel,
        out_type=jax.ShapeDtypeStruct((_NC * npad,), jnp.float32),
        mesh=mesh,
        scratch_types=[
            pltpu.VMEM((nch, _K), jnp.int32),
            pltpu.VMEM((16,), jnp.int32),
            pltpu.VMEM((_K,), jnp.float32),
            pltpu.VMEM((npad,), jnp.float32),
            pltpu.VMEM_SHARED((npad,), jnp.float32),
            pltpu.SemaphoreType.DMA,
            pltpu.SemaphoreType.DMA,
        ],
    )
    def body(dst_hbm, out_hbm, didx2, didx_t, ones, zbuf, acc, lsem, ssem):
        cid = lax.axis_index("c")
        sid = lax.axis_index("s")
        wid = sid * _NC + cid
        base = wid * epw
        for j in range(_K // 16):
            ones[pl.ds(j * 16, 16)] = jnp.full((16,), 1.0, jnp.float32)

        def lstep(i, c):
            pltpu.async_copy(dst_hbm.at[pl.ds(base + i * _K, _K)],
                             didx2.at[i], lsem)
            return c

        lax.fori_loop(0, nch, lstep, 0)

        @pl.when(sid == 0)
        def _zero():
            def zstep(i, c):
                zbuf[pl.ds(i * 16, 16)] = jnp.zeros((16,), jnp.float32)
                return c
            lax.fori_loop(0, npad // 16, zstep, 0)
            pltpu.sync_copy(zbuf, acc)

        def ldrain(i, c):
            pltpu.make_async_copy(dst_hbm.at[pl.ds(base, _K)], didx2.at[0],
                                  lsem).wait()
            return c

        lax.fori_loop(0, nch, ldrain, 0)
        plsc.subcore_barrier()

        def step(i, c):
            pltpu.async_copy(ones, acc.at[didx2.at[i]], ssem, add=True)
            return c

        lax.fori_loop(0, nch, step, 0)

        def sdrain(i, c):
            pltpu.make_async_copy(ones, acc.at[didx2.at[0]], ssem).wait()
            return c

        lax.fori_loop(0, nch, sdrain, 0)
        if tail:
            pltpu.sync_copy(dst_hbm.at[pl.ds(base + nch * _K, tail)], didx_t)
            pltpu.sync_copy(ones.at[pl.ds(0, tail)], acc.at[didx_t], add=True)
        plsc.subcore_barrier()

        @pl.when(sid == 0)
        def _out():
            pltpu.sync_copy(acc, out_hbm.at[pl.ds(cid * npad, npad)])

    return body(dst)


def _seg_sum_sc(g, src, dst, zeros):
    n, d = g.shape
    e = src.shape[0]
    epw = e // _NW
    nch = epw // _K
    tail = epw - nch * _K
    assert tail % 8 == 0 and tail <= 16
    rpt = (n // _NS) // 8 * 8
    otail = n - _NS * rpt
    mesh = plsc.VectorSubcoreMesh(core_axis_name="c", subcore_axis_name="s")

    @functools.partial(
        pl.kernel,
        out_type=jax.ShapeDtypeStruct((_NC, n, d), jnp.float32),
        mesh=mesh,
        scratch_types=(
            [pltpu.VMEM_SHARED((n, d), jnp.float32),
             pltpu.VMEM((16,), jnp.int32),
             pltpu.VMEM((16,), jnp.int32),
             pltpu.VMEM((16, d), jnp.float32)]
            + [pltpu.VMEM((_K,), jnp.int32) for _ in range(2 * _NIDX)]
            + [pltpu.VMEM((_K, d), jnp.float32) for _ in range(_NBUF)]
            + [pltpu.SemaphoreType.DMA for _ in range(_NBUF + _NIDX)]
        ),
    )
    def body(g_hbm, src_hbm, dst_hbm, z_hbm, out_hbm, acc, sidx_t, didx_t,
             rows_t, *bufs):
        sidx = bufs[:_NIDX]
        didx = bufs[_NIDX:2 * _NIDX]
        rows = bufs[2 * _NIDX:2 * _NIDX + _NBUF]
        gsem = bufs[2 * _NIDX + _NBUF:2 * _NIDX + 2 * _NBUF]
        isem = bufs[2 * _NIDX + 2 * _NBUF:]
        cid = lax.axis_index("c")
        sid = lax.axis_index("s")
        wid = sid * _NC + cid
        base = wid * epw

        for q in range(_NIDX):
            pltpu.sync_copy(src_hbm.at[pl.ds(base + q * _K, _K)], sidx[q])
            pltpu.sync_copy(dst_hbm.at[pl.ds(base + q * _K, _K)], didx[q])
        for b in range(_NBUF):
            pltpu.async_copy(g_hbm.at[sidx[b]], rows[b], gsem[b])
        pltpu.sync_copy(z_hbm.at[pl.ds(sid * rpt, rpt)],
                        acc.at[pl.ds(sid * rpt, rpt)])

        @pl.when(sid == 0)
        def _ztail():
            pltpu.sync_copy(z_hbm.at[pl.ds(_NS * rpt, otail)],
                            acc.at[pl.ds(_NS * rpt, otail)])

        plsc.subcore_barrier()

        nsb = nch // _NIDX
        rem = nch - nsb * _NIDX

        def chunk_step(j, q, b, prefetch_ok):
            pltpu.make_async_copy(g_hbm.at[sidx[q]], rows[b], gsem[b]).wait()
            pltpu.sync_copy(rows[b], acc.at[didx[q]], add=True)
            if prefetch_ok:
                @pl.when(j + _NIDX < nch)
                def _pfidx():
                    nxt = base + (j + _NIDX) * _K
                    pltpu.async_copy(src_hbm.at[pl.ds(nxt, _K)], sidx[q],
                                     isem[q])
                    pltpu.async_copy(dst_hbm.at[pl.ds(nxt, _K)], didx[q],
                                     isem[q])

                qn = (q + _NBUF) % _NIDX

                @pl.when(j + _NBUF < nch)
                def _next():
                    @pl.when(j + _NBUF >= _NIDX)
                    def _wait_idx():
                        nxt = base + (j + _NBUF) * _K
                        pltpu.make_async_copy(
                            src_hbm.at[pl.ds(nxt, _K)], sidx[qn],
                            isem[qn]).wait()
                        pltpu.make_async_copy(
                            dst_hbm.at[pl.ds(nxt, _K)], didx[qn],
                            isem[qn]).wait()

                    pltpu.async_copy(g_hbm.at[sidx[qn]], rows[b], gsem[b])

        def blk(k, c):
            j0 = k * _NIDX
            for q in range(_NIDX):
                chunk_step(j0 + q, q, q % _NBUF, True)
            return c

        lax.fori_loop(0, nsb, blk, 0)
        for r in range(rem):
            j = nsb * _NIDX + r
            chunk_step(j, j % _NIDX, j % _NBUF, False)
        if tail:
            tbase = base + nch * _K
            pltpu.sync_copy(src_hbm.at[pl.ds(tbase, tail)], sidx_t)
            pltpu.sync_copy(dst_hbm.at[pl.ds(tbase, tail)], didx_t)
            pltpu.sync_copy(g_hbm.at[sidx_t], rows_t)
            pltpu.sync_copy(rows_t, acc.at[didx_t], add=True)
        plsc.subcore_barrier()

        pltpu.sync_copy(acc.at[pl.ds(sid * rpt, rpt)],
                        out_hbm.at[cid, pl.ds(sid * rpt, rpt)])

        @pl.when(sid == 0)
        def _otail():
            pltpu.sync_copy(acc.at[pl.ds(_NS * rpt, otail)],
                            out_hbm.at[cid, pl.ds(_NS * rpt, otail)])

    return body(g, src, dst, zeros)


def _tc1(x, w1, deg_t):
    n, d = x.shape

    def body(x_ref, w_ref, deg_ref, g1_ref, dinv_ref):
        deg = deg_ref[:, 0:1] + deg_ref[:, 1:2] + 1.0
        dinv = lax.rsqrt(deg)
        h = jnp.dot(x_ref[...], w_ref[...], preferred_element_type=jnp.float32)
        g1_ref[...] = h * dinv
        dinv_ref[...] = dinv

    return pl.pallas_call(
        body,
        out_shape=(jax.ShapeDtypeStruct((n, d), jnp.float32),
                   jax.ShapeDtypeStruct((n, 1), jnp.float32)),
    )(x, w1, deg_t)


def _tc2(s1p, g1, dinv, b1, gnw, gnb, gna, w2):
    n, d = g1.shape

    def body(sp_ref, g_ref, di_ref, b_ref, w_ref, bt_ref, a_ref, w2_ref,
             x1_ref, g2_ref):
        s = sp_ref[0] + sp_ref[1] + g_ref[...]
        y = di_ref[...] * s + b_ref[...]
        mean = jnp.mean(y, axis=0, keepdims=True)
        o = y - a_ref[...] * mean
        var = jnp.mean(o * o, axis=0, keepdims=True)
        x1 = jnp.maximum(w_ref[...] * o * lax.rsqrt(var + _EPS) + bt_ref[...],
                         0.0)
        x1_ref[...] = x1
        g2_ref[...] = jnp.dot(x1, w2_ref[...],
                              preferred_element_type=jnp.float32) * di_ref[...]

    return pl.pallas_call(
        body,
        out_shape=(jax.ShapeDtypeStruct((n, d), jnp.float32),
                   jax.ShapeDtypeStruct((n, d), jnp.float32)),
    )(s1p, g1, dinv, b1, gnw, gnb, gna, w2)


def _tc3(s2p, g2, dinv, b2, gnw, gnb, gna, x1, wr, br):
    n, d = g2.shape

    def body(sp_ref, g_ref, di_ref, b_ref, w_ref, bt_ref, a_ref, x1_ref,
             wr_ref, br_ref, out_ref):
        s = sp_ref[0] + sp_ref[1] + g_ref[...]
        y = di_ref[...] * s + b_ref[...]
        mean = jnp.mean(y, axis=0, keepdims=True)
        o = y - a_ref[...] * mean
        var = jnp.mean(o * o, axis=0, keepdims=True)
        x2 = jnp.maximum(w_ref[...] * o * lax.rsqrt(var + _EPS) + bt_ref[...],
                         0.0)
        out_ref[...] = jnp.dot(x1_ref[...] + x2, wr_ref[...],
                               preferred_element_type=jnp.float32) + br_ref[...]

    return pl.pallas_call(
        body,
        out_shape=jax.ShapeDtypeStruct((n, d), jnp.float32),
    )(s2p, g2, dinv, b2, gnw, gnb, gna, x1, wr, br)


def kernel(x, edge_index, W1, b1, W2, b2, gn1_w, gn1_b, gn1_a, gn2_w, gn2_b,
           gn2_a, Wr, br):
    n, d = x.shape
    e = edge_index.shape[1]
    assert e % _NW == 0 and n % 16 == 0

    src = edge_index[0]
    dst = edge_index[1]
    zeros = jnp.zeros((n, d), jnp.float32)

    npad = -(-n // 128) * 128
    degp = _deg_sc(dst, n, npad).reshape(_NC, npad)[:, :n]
    deg_t = degp.T

    g1, dinv = _tc1(x, W1, deg_t)
    s1p = _seg_sum_sc(g1, src, dst, zeros)
    x1, g2 = _tc2(s1p, g1, dinv, b1.reshape(1, d), gn1_w.reshape(1, d),
                  gn1_b.reshape(1, d), gn1_a.reshape(1, d), W2)
    s2p = _seg_sum_sc(g2, src, dst, zeros)
    return _tc3(s2p, g2, dinv, b2.reshape(1, d), gn2_w.reshape(1, d),
                gn2_b.reshape(1, d), gn2_a.reshape(1, d), x1, Wr, br.reshape(1, d))


# speedup vs baseline: 3.7245x; 1.0147x over previous
import functools

import jax
import jax.numpy as jnp
from jax import lax
from jax.experimental import pallas as pl
from jax.experimental.pallas import tpu as pltpu
from jax.experimental.pallas import tpu_sc as plsc

_EPS = 1e-5
_NC = 2
_NS = 16
_NW = _NC * _NS
_K = 128
_NBUF = 2
_NIDX = 4


def _deg_sc(dst, n, npad):
    e = dst.shape[0]
    epw = e // _NW
    nch = epw // _K
    tail = epw - nch * _K
    mesh = plsc.VectorSubcoreMesh(core_axis_name="c", subcore_axis_name="s")

    @functools.partial(
        pl.kernel,
        out_type=jax.ShapeDtypeStruct((_NC * npad,), jnp.float32),
        mesh=mesh,
        scratch_types=[
            pltpu.VMEM((nch, _K), jnp.int32),
            pltpu.VMEM((16,), jnp.int32),
            pltpu.VMEM((_K,), jnp.float32),
            pltpu.VMEM((npad,), jnp.float32),
            pltpu.VMEM_SHARED((npad,), jnp.float32),
            pltpu.SemaphoreType.DMA,
            pltpu.SemaphoreType.DMA,
        ],
    )
    def body(dst_hbm, out_hbm, didx2, didx_t, ones, zbuf, acc, lsem, ssem):
        cid = lax.axis_index("c")
        sid = lax.axis_index("s")
        wid = sid * _NC + cid
        base = wid * epw
        for j in range(_K // 16):
            ones[pl.ds(j * 16, 16)] = jnp.full((16,), 1.0, jnp.float32)

        def lstep(i, c):
            pltpu.async_copy(dst_hbm.at[pl.ds(base + i * _K, _K)],
                             didx2.at[i], lsem)
            return c

        lax.fori_loop(0, nch, lstep, 0)

        @pl.when(sid == 0)
        def _zero():
            def zstep(i, c):
                zbuf[pl.ds(i * 16, 16)] = jnp.zeros((16,), jnp.float32)
                return c
            lax.fori_loop(0, npad // 16, zstep, 0)
            pltpu.sync_copy(zbuf, acc)

        def ldrain(i, c):
            pltpu.make_async_copy(dst_hbm.at[pl.ds(base, _K)], didx2.at[0],
                                  lsem).wait()
            return c

        lax.fori_loop(0, nch, ldrain, 0)
        plsc.subcore_barrier()

        def step(i, c):
            pltpu.async_copy(ones, acc.at[didx2.at[i]], ssem, add=True)
            return c

        lax.fori_loop(0, nch, step, 0)

        def sdrain(i, c):
            pltpu.make_async_copy(ones, acc.at[didx2.at[0]], ssem).wait()
            return c

        lax.fori_loop(0, nch, sdrain, 0)
        if tail:
            pltpu.sync_copy(dst_hbm.at[pl.ds(base + nch * _K, tail)], didx_t)
            pltpu.sync_copy(ones.at[pl.ds(0, tail)], acc.at[didx_t], add=True)
        plsc.subcore_barrier()

        @pl.when(sid == 0)
        def _out():
            pltpu.sync_copy(acc, out_hbm.at[pl.ds(cid * npad, npad)])

    return body(dst)


def _seg_sum_sc(g, src, dst, zeros):
    n, d = g.shape
    e = src.shape[0]
    epw = e // _NW
    nch = epw // _K
    tail = epw - nch * _K
    assert tail % 8 == 0 and tail <= 16
    rpt = (n // _NS) // 8 * 8
    otail = n - _NS * rpt
    mesh = plsc.VectorSubcoreMesh(core_axis_name="c", subcore_axis_name="s")

    @functools.partial(
        pl.kernel,
        out_type=jax.ShapeDtypeStruct((_NC, n, d), jnp.float32),
        mesh=mesh,
        scratch_types=(
            [pltpu.VMEM_SHARED((n, d), jnp.float32),
             pltpu.VMEM((16,), jnp.int32),
             pltpu.VMEM((16,), jnp.int32),
             pltpu.VMEM((16, d), jnp.float32)]
            + [pltpu.VMEM((_K,), jnp.int32) for _ in range(2 * _NIDX)]
            + [pltpu.VMEM((_K, d), jnp.float32) for _ in range(_NBUF)]
            + [pltpu.SemaphoreType.DMA for _ in range(_NBUF + _NIDX)]
        ),
    )
    def body(g_hbm, src_hbm, dst_hbm, z_hbm, out_hbm, acc, sidx_t, didx_t,
             rows_t, *bufs):
        sidx = bufs[:_NIDX]
        didx = bufs[_NIDX:2 * _NIDX]
        rows = bufs[2 * _NIDX:2 * _NIDX + _NBUF]
        gsem = bufs[2 * _NIDX + _NBUF:2 * _NIDX + 2 * _NBUF]
        isem = bufs[2 * _NIDX + 2 * _NBUF:]
        cid = lax.axis_index("c")
        sid = lax.axis_index("s")
        wid = sid * _NC + cid
        base = wid * epw

        for q in range(_NIDX):
            pltpu.sync_copy(src_hbm.at[pl.ds(base + q * _K, _K)], sidx[q])
            pltpu.sync_copy(dst_hbm.at[pl.ds(base + q * _K, _K)], didx[q])
        for b in range(_NBUF):
            pltpu.async_copy(g_hbm.at[sidx[b]], rows[b], gsem[b])
        pltpu.sync_copy(z_hbm.at[pl.ds(sid * rpt, rpt)],
                        acc.at[pl.ds(sid * rpt, rpt)])

        @pl.when(sid == 0)
        def _ztail():
            pltpu.sync_copy(z_hbm.at[pl.ds(_NS * rpt, otail)],
                            acc.at[pl.ds(_NS * rpt, otail)])

        plsc.subcore_barrier()

        nsb = nch // _NIDX
        rem = nch - nsb * _NIDX

        def chunk_step(j, q, b, prefetch_ok):
            pltpu.make_async_copy(g_hbm.at[sidx[q]], rows[b], gsem[b]).wait()
            pltpu.sync_copy(rows[b], acc.at[didx[q]], add=True)
            if prefetch_ok:
                @pl.when(j + _NIDX < nch)
                def _pfidx():
                    nxt = base + (j + _NIDX) * _K
                    pltpu.async_copy(src_hbm.at[pl.ds(nxt, _K)], sidx[q],
                                     isem[q])
                    pltpu.async_copy(dst_hbm.at[pl.ds(nxt, _K)], didx[q],
                                     isem[q])

                qn = (q + _NBUF) % _NIDX

                @pl.when(j + _NBUF < nch)
                def _next():
                    @pl.when(j + _NBUF >= _NIDX)
                    def _wait_idx():
                        nxt = base + (j + _NBUF) * _K
                        pltpu.make_async_copy(
                            src_hbm.at[pl.ds(nxt, _K)], sidx[qn],
                            isem[qn]).wait()
                        pltpu.make_async_copy(
                            dst_hbm.at[pl.ds(nxt, _K)], didx[qn],
                            isem[qn]).wait()

                    pltpu.async_copy(g_hbm.at[sidx[qn]], rows[b], gsem[b])

        def blk(k, c):
            j0 = k * _NIDX
            for q in range(_NIDX):
                chunk_step(j0 + q, q, q % _NBUF, True)
            return c

        lax.fori_loop(0, nsb, blk, 0)
        for r in range(rem):
            j = nsb * _NIDX + r
            chunk_step(j, j % _NIDX, j % _NBUF, False)
        if tail:
            tbase = base + nch * _K
            pltpu.sync_copy(src_hbm.at[pl.ds(tbase, tail)], sidx_t)
            pltpu.sync_copy(dst_hbm.at[pl.ds(tbase, tail)], didx_t)
            pltpu.sync_copy(g_hbm.at[sidx_t], rows_t)
            pltpu.sync_copy(rows_t, acc.at[didx_t], add=True)
        plsc.subcore_barrier()

        pltpu.sync_copy(acc.at[pl.ds(sid * rpt, rpt)],
                        out_hbm.at[cid, pl.ds(sid * rpt, rpt)])

        @pl.when(sid == 0)
        def _otail():
            pltpu.sync_copy(acc.at[pl.ds(_NS * rpt, otail)],
                            out_hbm.at[cid, pl.ds(_NS * rpt, otail)])

    return body(g, src, dst, zeros)


def _dinv_col(deg_ref, n):
    deg_row = deg_ref[0:1, :] + deg_ref[1:2, :] + 1.0
    dinv_row = lax.rsqrt(deg_row)
    return jnp.transpose(dinv_row, (1, 0))[0:n, :]


def _tc_mm(x, w1):
    n, d = x.shape

    def body(x_ref, w_ref, h_ref):
        h_ref[...] = jnp.dot(x_ref[...], w_ref[...],
                             preferred_element_type=jnp.float32)

    return pl.pallas_call(
        body, out_shape=jax.ShapeDtypeStruct((n, d), jnp.float32))(x, w1)


def _tc_scale(h, degc):
    n, d = h.shape

    def body(h_ref, deg_ref, g1_ref):
        g1_ref[...] = h_ref[...] * _dinv_col(deg_ref, n)

    return pl.pallas_call(
        body, out_shape=jax.ShapeDtypeStruct((n, d), jnp.float32))(h, degc)


def _tc2(s1p, g1, degc, b1, gnw, gnb, gna, w2):
    n, d = g1.shape

    def body(sp_ref, g_ref, deg_ref, b_ref, w_ref, bt_ref, a_ref, w2_ref,
             x1_ref, g2_ref):
        dinv = _dinv_col(deg_ref, n)
        s = sp_ref[0] + sp_ref[1] + g_ref[...]
        y = dinv * s + b_ref[...]
        mean = jnp.mean(y, axis=0, keepdims=True)
        o = y - a_ref[...] * mean
        var = jnp.mean(o * o, axis=0, keepdims=True)
        x1 = jnp.maximum(w_ref[...] * o * lax.rsqrt(var + _EPS) + bt_ref[...],
                         0.0)
        x1_ref[...] = x1
        g2_ref[...] = jnp.dot(x1, w2_ref[...],
                              preferred_element_type=jnp.float32) * dinv

    return pl.pallas_call(
        body,
        out_shape=(jax.ShapeDtypeStruct((n, d), jnp.float32),
                   jax.ShapeDtypeStruct((n, d), jnp.float32)),
    )(s1p, g1, degc, b1, gnw, gnb, gna, w2)


def _tc3(s2p, g2, degc, b2, gnw, gnb, gna, x1, wr, br):
    n, d = g2.shape

    def body(sp_ref, g_ref, deg_ref, b_ref, w_ref, bt_ref, a_ref, x1_ref,
             wr_ref, br_ref, out_ref):
        s = sp_ref[0] + sp_ref[1] + g_ref[...]
        y = _dinv_col(deg_ref, n) * s + b_ref[...]
        mean = jnp.mean(y, axis=0, keepdims=True)
        o = y - a_ref[...] * mean
        var = jnp.mean(o * o, axis=0, keepdims=True)
        x2 = jnp.maximum(w_ref[...] * o * lax.rsqrt(var + _EPS) + bt_ref[...],
                         0.0)
        out_ref[...] = jnp.dot(x1_ref[...] + x2, wr_ref[...],
                               preferred_element_type=jnp.float32) + br_ref[...]

    return pl.pallas_call(
        body,
        out_shape=jax.ShapeDtypeStruct((n, d), jnp.float32),
    )(s2p, g2, degc, b2, gnw, gnb, gna, x1, wr, br)


def kernel(x, edge_index, W1, b1, W2, b2, gn1_w, gn1_b, gn1_a, gn2_w, gn2_b,
           gn2_a, Wr, br):
    n, d = x.shape
    e = edge_index.shape[1]
    assert e % _NW == 0 and n % 16 == 0

    src = edge_index[0]
    dst = edge_index[1]
    zeros = jnp.zeros((n, d), jnp.float32)

    npad = -(-n // 128) * 128
    h1 = _tc_mm(x, W1)
    degc = _deg_sc(dst, n, npad).reshape(_NC, npad)

    g1 = _tc_scale(h1, degc)
    s1p = _seg_sum_sc(g1, src, dst, zeros)
    x1, g2 = _tc2(s1p, g1, degc, b1.reshape(1, d), gn1_w.reshape(1, d),
                  gn1_b.reshape(1, d), gn1_a.reshape(1, d), W2)
    s2p = _seg_sum_sc(g2, src, dst, zeros)
    return _tc3(s2p, g2, degc, b2.reshape(1, d), gn2_w.reshape(1, d),
                gn2_b.reshape(1, d), gn2_a.reshape(1, d), x1, Wr, br.reshape(1, d))
